# baseline (speedup 1.0000x reference)
import functools

import jax
import jax.numpy as jnp
from jax import lax
from jax.experimental import pallas as pl
from jax.experimental.pallas import tpu as pltpu
from jax.experimental.pallas import tpu_sc as plsc

N = 10000
E = 320000
D = 128

NC = 2
NS = 16
NW = NC * NS

CH = 128
NCHUNK = 80
EPT = CH * NCHUNK
E_PAD = EPT * NW

AGG_ROWS = 10112
AGG_PT = AGG_ROWS // NS
W0 = 80
W1 = 80
NW1 = NS * W1
NCHT = NS * (W0 + W1)
DEG_PAD = 10240
DEG_PT = DEG_PAD // NS

_mesh = plsc.VectorSubcoreMesh(core_axis_name="c", subcore_axis_name="s")


@functools.partial(
    pl.kernel,
    out_type=jax.ShapeDtypeStruct((NC, DEG_PAD), jnp.float32),
    mesh=_mesh,
    scratch_types=[
        pltpu.VMEM((NCHUNK, CH), jnp.int32),
        pltpu.VMEM((CH,), jnp.float32),
        pltpu.VMEM((DEG_PT,), jnp.float32),
        pltpu.VMEM_SHARED((DEG_PAD,), jnp.float32),
        pltpu.SemaphoreType.DMA,
    ],
)
def _deg_kernel(dst_hbm, out_hbm, dst_v, ones_v, zer_v, deg_sh, sem):
    cid = lax.axis_index("c")
    sid = lax.axis_index("s")
    wid = sid * NC + cid
    for i in range(CH // 16):
        ones_v[pl.ds(i * 16, 16)] = jnp.ones((16,), jnp.float32)
    for i in range(DEG_PT // 16):
        zer_v[pl.ds(i * 16, 16)] = jnp.zeros((16,), jnp.float32)
    pltpu.sync_copy(zer_v, deg_sh.at[pl.ds(sid * DEG_PT, DEG_PT)])
    pltpu.sync_copy(dst_hbm.at[wid], dst_v)
    plsc.subcore_barrier()

    def fire(i, carry):
        pltpu.async_copy(ones_v, deg_sh.at[dst_v.at[i]], sem, add=True)
        return carry

    lax.fori_loop(0, NCHUNK, fire, 0)

    def drain(i, carry):
        pltpu.make_async_copy(ones_v, deg_sh.at[dst_v.at[i]], sem).wait()
        return carry

    lax.fori_loop(0, NCHUNK, drain, 0)
    plsc.subcore_barrier()
    pltpu.sync_copy(
        deg_sh.at[pl.ds(sid * DEG_PT, DEG_PT)],
        out_hbm.at[cid, pl.ds(sid * DEG_PT, DEG_PT)],
    )


@functools.partial(
    pl.kernel,
    out_type=jax.ShapeDtypeStruct((NC, AGG_ROWS, D), jnp.float32),
    mesh=_mesh,
    scratch_types=[
        pltpu.VMEM((W0, CH), jnp.int32),
        pltpu.VMEM((2, CH), jnp.int32),
        pltpu.VMEM((CH, D), jnp.float32),
        pltpu.VMEM((CH, D), jnp.float32),
        pltpu.VMEM_SHARED((AGG_ROWS, D), jnp.float32),
        pltpu.SemaphoreType.DMA,
        pltpu.SemaphoreType.DMA,
        pltpu.SemaphoreType.DMA,
        pltpu.SemaphoreType.DMA,
        pltpu.SemaphoreType.DMA,
        pltpu.SemaphoreType.DMA,
    ],
)
def _agg_kernel(hs_hbm, src_hbm, dst_hbm, zeros_hbm, out_hbm,
                src_v, dst_v, r0, r1, agg_sh,
                sem0, sem1, semd0, semd1, sems0, sems1):
    cid = lax.axis_index("c")
    sid = lax.axis_index("s")
    base = jnp.where(cid == 0, NW1 + sid * W0, sid * W1)
    nch = jnp.where(cid == 0, W0, W1)
    pltpu.sync_copy(zeros_hbm, agg_sh.at[pl.ds(sid * AGG_PT, AGG_PT)])

    @pl.when(cid == 0)
    def _():
        pltpu.sync_copy(src_hbm.at[pl.ds(NW1 + sid * W0, W0)],
                        src_v.at[pl.ds(0, W0)])

    @pl.when(cid == 1)
    def _():
        pltpu.sync_copy(src_hbm.at[pl.ds(sid * W1, W1)],
                        src_v.at[pl.ds(0, W1)])

    plsc.subcore_barrier()

    pltpu.async_copy(hs_hbm.at[src_v.at[0]], r0, sem0)
    pltpu.async_copy(hs_hbm.at[src_v.at[1]], r1, sem1)
    pltpu.async_copy(dst_hbm.at[base], dst_v.at[0], semd0)
    pltpu.async_copy(dst_hbm.at[base + 1], dst_v.at[1], semd1)

    def pair(i, carry):
        c0 = 2 * i
        pltpu.make_async_copy(hs_hbm.at[src_v.at[c0]], r0, sem0).wait()
        pltpu.make_async_copy(dst_hbm.at[base + c0], dst_v.at[0],
                              semd0).wait()
        pltpu.async_copy(r0, agg_sh.at[dst_v.at[0]], sems0, add=True)
        pltpu.make_async_copy(hs_hbm.at[src_v.at[c0 + 1]], r1, sem1).wait()
        pltpu.make_async_copy(dst_hbm.at[base + c0 + 1], dst_v.at[1],
                              semd1).wait()
        pltpu.async_copy(r1, agg_sh.at[dst_v.at[1]], sems1, add=True)

        @pl.when(c0 + 2 < nch)
        def _():
            pltpu.make_async_copy(r0, agg_sh.at[dst_v.at[0]], sems0).wait()
            pltpu.async_copy(hs_hbm.at[src_v.at[c0 + 2]], r0, sem0)
            pltpu.async_copy(dst_hbm.at[base + c0 + 2], dst_v.at[0], semd0)

        @pl.when(c0 + 3 < nch)
        def _():
            pltpu.make_async_copy(r1, agg_sh.at[dst_v.at[1]], sems1).wait()
            pltpu.async_copy(hs_hbm.at[src_v.at[c0 + 3]], r1, sem1)
            pltpu.async_copy(dst_hbm.at[base + c0 + 3], dst_v.at[1], semd1)

        return carry

    lax.fori_loop(0, nch // 2, pair, 0)
    pltpu.make_async_copy(r0, agg_sh.at[dst_v.at[0]], sems0).wait()
    pltpu.make_async_copy(r1, agg_sh.at[dst_v.at[1]], sems1).wait()
    plsc.subcore_barrier()
    pltpu.sync_copy(
        agg_sh.at[pl.ds(sid * AGG_PT, AGG_PT)],
        out_hbm.at[cid, pl.ds(sid * AGG_PT, AGG_PT)],
    )


def _spow(d_a, e):
    return jnp.where(
        d_a > 0.0, jnp.exp(e * jnp.log(jnp.maximum(d_a, 1e-12))), 0.0
    )


def _l1_body(g1_ref, x_ref, w1_ref, b1_ref, dega_ref, degb_ref, h_ref, hs_ref):
    h = jnp.dot(x_ref[...], w1_ref[...], preferred_element_type=jnp.float32)
    h = h + b1_ref[...]
    a = g1_ref[6]
    d_a = dega_ref[...] + degb_ref[...] + a
    h_ref[...] = h
    hs_ref[...] = h * _spow(d_a, g1_ref[5])


def _mid_body(g1_ref, g2_ref, h1_ref, hs1_ref, a1a_ref, a1b_ref,
              dega_ref, degb_ref, w2_ref, b2_ref, h2_ref, hs2_ref):
    m1, m2, m3 = g1_ref[0], g1_ref[1], g1_ref[2]
    e1, e2, a = g1_ref[3], g1_ref[4], g1_ref[6]
    deg = dega_ref[...] + degb_ref[...]
    d_a = deg + a
    h1 = h1_ref[...]
    agg = a1a_ref[...] + a1b_ref[...] + a * hs1_ref[...]
    out1 = m1 * _spow(d_a, e1) * h1 + m2 * (_spow(d_a, e2) * agg) + m3 * h1
    r = jnp.maximum(out1, 0.0)
    h2 = jnp.dot(r, w2_ref[...], preferred_element_type=jnp.float32)
    h2 = h2 + b2_ref[...]
    d_a2 = deg + g2_ref[6]
    h2_ref[...] = h2
    hs2_ref[...] = h2 * _spow(d_a2, g2_ref[5])


def _fin_body(g2_ref, h2_ref, hs2_ref, a2a_ref, a2b_ref,
              dega_ref, degb_ref, out_ref):
    m1, m2, m3 = g2_ref[0], g2_ref[1], g2_ref[2]
    e1, e2, a = g2_ref[3], g2_ref[4], g2_ref[6]
    d_a = dega_ref[...] + degb_ref[...] + a
    h2 = h2_ref[...]
    agg = a2a_ref[...] + a2b_ref[...] + a * hs2_ref[...]
    out_ref[...] = (
        m1 * _spow(d_a, e1) * h2 + m2 * (_spow(d_a, e2) * agg) + m3 * h2
    )


_B = 1000
_GRID = N // _B

_smem = pl.BlockSpec(memory_space=pltpu.SMEM)


def _row_spec(width=D):
    return pl.BlockSpec((_B, width), lambda i: (i, 0))


def _full_spec(shape):
    return pl.BlockSpec(shape, lambda i: (0, 0))


_l1_call = pl.pallas_call(
    _l1_body,
    grid=(_GRID,),
    in_specs=[
        _smem,
        _row_spec(),
        _full_spec((D, D)),
        _full_spec((1, D)),
        _row_spec(1),
        _row_spec(1),
    ],
    out_specs=[_row_spec(), _row_spec()],
    out_shape=[
        jax.ShapeDtypeStruct((N, D), jnp.float32),
        jax.ShapeDtypeStruct((N, D), jnp.float32),
    ],
)

_mid_call = pl.pallas_call(
    _mid_body,
    grid=(_GRID,),
    in_specs=[
        _smem,
        _smem,
        _row_spec(),
        _row_spec(),
        _row_spec(),
        _row_spec(),
        _row_spec(1),
        _row_spec(1),
        _full_spec((D, D)),
        _full_spec((1, D)),
    ],
    out_specs=[_row_spec(), _row_spec()],
    out_shape=[
        jax.ShapeDtypeStruct((N, D), jnp.float32),
        jax.ShapeDtypeStruct((N, D), jnp.float32),
    ],
)

_fin_call = pl.pallas_call(
    _fin_body,
    grid=(_GRID,),
    in_specs=[
        _smem,
        _row_spec(),
        _row_spec(),
        _row_spec(),
        _row_spec(),
        _row_spec(1),
        _row_spec(1),
    ],
    out_specs=_row_spec(),
    out_shape=jax.ShapeDtypeStruct((N, D), jnp.float32),
)


@jax.jit
def kernel(x, W1, b1, W2, b2, g1, g2, edge_index):
    src = edge_index[0].astype(jnp.int32)
    dst = edge_index[1].astype(jnp.int32)
    pad = E_PAD - E
    pidx = jnp.arange(pad, dtype=jnp.int32)
    srcp = jnp.concatenate([src, pidx % N])
    dstp = jnp.concatenate([dst, N + pidx % (AGG_ROWS - N)])
    dst_deg = dstp.reshape(NW, NCHUNK, CH)
    src_agg = srcp.reshape(NCHT, CH)
    dst_agg = dstp.reshape(NCHT, CH)

    degp = _deg_kernel(dst_deg)
    dega = degp[0, :N, None]
    degb = degp[1, :N, None]

    zeros_agg = jnp.zeros((AGG_PT, D), jnp.float32)
    b1r = b1.reshape(1, D)
    b2r = b2.reshape(1, D)

    h1, hs1 = _l1_call(g1, x, W1, b1r, dega, degb)
    agg1 = _agg_kernel(hs1, src_agg, dst_agg, zeros_agg)
    h2, hs2 = _mid_call(g1, g2, h1, hs1, agg1[0, :N], agg1[1, :N],
                        dega, degb, W2, b2r)
    agg2 = _agg_kernel(hs2, src_agg, dst_agg, zeros_agg)
    return _fin_call(g2, h2, hs2, agg2[0, :N], agg2[1, :N], dega, degb)


# speedup vs baseline: 1.2394x; 1.2394x over previous
import functools

import jax
import jax.numpy as jnp
from jax import lax
from jax.experimental import pallas as pl
from jax.experimental.pallas import tpu as pltpu
from jax.experimental.pallas import tpu_sc as plsc

N = 10000
E = 320000
D = 128

NC = 2
NS = 16
NW = NC * NS

CH = 128
NCHUNK = 80
EPT = CH * NCHUNK
E_PAD = EPT * NW

AGG_ROWS = 10112
AGG_PT = AGG_ROWS // NS
W0 = 80
W1 = 80
NW1 = NS * W1
NCHT = NS * (W0 + W1)
DEG_PAD = 10240
DEG_PT = DEG_PAD // NS

_mesh = plsc.VectorSubcoreMesh(core_axis_name="c", subcore_axis_name="s")


@functools.partial(
    pl.kernel,
    out_type=jax.ShapeDtypeStruct((NC, DEG_PAD), jnp.float32),
    mesh=_mesh,
    scratch_types=[
        pltpu.VMEM((NCHUNK, CH), jnp.int32),
        pltpu.VMEM((CH,), jnp.float32),
        pltpu.VMEM((DEG_PT,), jnp.float32),
        pltpu.VMEM_SHARED((DEG_PAD,), jnp.float32),
        pltpu.SemaphoreType.DMA,
    ],
)
def _deg_kernel(dst_hbm, out_hbm, dst_v, ones_v, zer_v, deg_sh, sem):
    cid = lax.axis_index("c")
    sid = lax.axis_index("s")
    wid = sid * NC + cid
    for i in range(CH // 16):
        ones_v[pl.ds(i * 16, 16)] = jnp.ones((16,), jnp.float32)
    for i in range(DEG_PT // 16):
        zer_v[pl.ds(i * 16, 16)] = jnp.zeros((16,), jnp.float32)
    pltpu.sync_copy(zer_v, deg_sh.at[pl.ds(sid * DEG_PT, DEG_PT)])
    pltpu.sync_copy(dst_hbm.at[wid], dst_v)
    plsc.subcore_barrier()

    def fire(i, carry):
        pltpu.async_copy(ones_v, deg_sh.at[dst_v.at[i]], sem, add=True)
        return carry

    lax.fori_loop(0, NCHUNK, fire, 0)

    def drain(i, carry):
        pltpu.make_async_copy(ones_v, deg_sh.at[dst_v.at[i]], sem).wait()
        return carry

    lax.fori_loop(0, NCHUNK, drain, 0)
    plsc.subcore_barrier()
    pltpu.sync_copy(
        deg_sh.at[pl.ds(sid * DEG_PT, DEG_PT)],
        out_hbm.at[cid, pl.ds(sid * DEG_PT, DEG_PT)],
    )


@functools.partial(
    pl.kernel,
    out_type=jax.ShapeDtypeStruct((NC, AGG_ROWS, D), jnp.float32),
    mesh=_mesh,
    scratch_types=[
        pltpu.VMEM((W0, CH), jnp.int32),
        pltpu.VMEM((2, CH), jnp.int32),
        pltpu.VMEM((CH, D), jnp.float32),
        pltpu.VMEM((CH, D), jnp.float32),
        pltpu.VMEM_SHARED((AGG_ROWS, D), jnp.float32),
        pltpu.SemaphoreType.DMA,
        pltpu.SemaphoreType.DMA,
        pltpu.SemaphoreType.DMA,
        pltpu.SemaphoreType.DMA,
        pltpu.SemaphoreType.DMA,
        pltpu.SemaphoreType.DMA,
    ],
)
def _agg_kernel(hs_hbm, src_hbm, dst_hbm, zeros_hbm, out_hbm,
                src_v, dst_v, r0, r1, agg_sh,
                sem0, sem1, semd0, semd1, sems0, sems1):
    cid = lax.axis_index("c")
    sid = lax.axis_index("s")
    base = jnp.where(cid == 0, NW1 + sid * W0, sid * W1)
    nch = jnp.where(cid == 0, W0, W1)
    pltpu.sync_copy(zeros_hbm, agg_sh.at[pl.ds(sid * AGG_PT, AGG_PT)])

    @pl.when(cid == 0)
    def _():
        pltpu.sync_copy(src_hbm.at[pl.ds(NW1 + sid * W0, W0)],
                        src_v.at[pl.ds(0, W0)])

    @pl.when(cid == 1)
    def _():
        pltpu.sync_copy(src_hbm.at[pl.ds(sid * W1, W1)],
                        src_v.at[pl.ds(0, W1)])

    plsc.subcore_barrier()

    pltpu.async_copy(hs_hbm.at[src_v.at[0]], r0, sem0)
    pltpu.async_copy(dst_hbm.at[base], dst_v.at[0], semd0)

    def pair(i, carry):
        c0 = 2 * i
        pltpu.async_copy(hs_hbm.at[src_v.at[c0 + 1]], r1, sem1)
        pltpu.async_copy(dst_hbm.at[base + c0 + 1], dst_v.at[1], semd1)
        pltpu.make_async_copy(hs_hbm.at[src_v.at[c0]], r0, sem0).wait()
        pltpu.make_async_copy(dst_hbm.at[base + c0], dst_v.at[0],
                              semd0).wait()
        pltpu.sync_copy(r0, agg_sh.at[dst_v.at[0]], add=True)

        @pl.when(c0 + 2 < nch)
        def _():
            pltpu.async_copy(hs_hbm.at[src_v.at[c0 + 2]], r0, sem0)
            pltpu.async_copy(dst_hbm.at[base + c0 + 2], dst_v.at[0], semd0)

        pltpu.make_async_copy(hs_hbm.at[src_v.at[c0 + 1]], r1, sem1).wait()
        pltpu.make_async_copy(dst_hbm.at[base + c0 + 1], dst_v.at[1],
                              semd1).wait()
        pltpu.sync_copy(r1, agg_sh.at[dst_v.at[1]], add=True)
        return carry

    lax.fori_loop(0, nch // 2, pair, 0)
    plsc.subcore_barrier()
    pltpu.sync_copy(
        agg_sh.at[pl.ds(sid * AGG_PT, AGG_PT)],
        out_hbm.at[cid, pl.ds(sid * AGG_PT, AGG_PT)],
    )


def _spow(d_a, e):
    return jnp.where(
        d_a > 0.0, jnp.exp(e * jnp.log(jnp.maximum(d_a, 1e-12))), 0.0
    )


def _l1_body(g1_ref, x_ref, w1_ref, b1_ref, dega_ref, degb_ref, h_ref, hs_ref):
    h = jnp.dot(x_ref[...], w1_ref[...], preferred_element_type=jnp.float32)
    h = h + b1_ref[...]
    a = g1_ref[6]
    d_a = dega_ref[...] + degb_ref[...] + a
    h_ref[...] = h
    hs_ref[...] = h * _spow(d_a, g1_ref[5])


def _mid_body(g1_ref, g2_ref, h1_ref, hs1_ref, a1a_ref, a1b_ref,
              dega_ref, degb_ref, w2_ref, b2_ref, h2_ref, hs2_ref):
    m1, m2, m3 = g1_ref[0], g1_ref[1], g1_ref[2]
    e1, e2, a = g1_ref[3], g1_ref[4], g1_ref[6]
    deg = dega_ref[...] + degb_ref[...]
    d_a = deg + a
    h1 = h1_ref[...]
    agg = a1a_ref[...] + a1b_ref[...] + a * hs1_ref[...]
    out1 = m1 * _spow(d_a, e1) * h1 + m2 * (_spow(d_a, e2) * agg) + m3 * h1
    r = jnp.maximum(out1, 0.0)
    h2 = jnp.dot(r, w2_ref[...], preferred_element_type=jnp.float32)
    h2 = h2 + b2_ref[...]
    d_a2 = deg + g2_ref[6]
    h2_ref[...] = h2
    hs2_ref[...] = h2 * _spow(d_a2, g2_ref[5])


def _fin_body(g2_ref, h2_ref, hs2_ref, a2a_ref, a2b_ref,
              dega_ref, degb_ref, out_ref):
    m1, m2, m3 = g2_ref[0], g2_ref[1], g2_ref[2]
    e1, e2, a = g2_ref[3], g2_ref[4], g2_ref[6]
    d_a = dega_ref[...] + degb_ref[...] + a
    h2 = h2_ref[...]
    agg = a2a_ref[...] + a2b_ref[...] + a * hs2_ref[...]
    out_ref[...] = (
        m1 * _spow(d_a, e1) * h2 + m2 * (_spow(d_a, e2) * agg) + m3 * h2
    )


_B = 1000
_GRID = N // _B

_smem = pl.BlockSpec(memory_space=pltpu.SMEM)


def _row_spec(width=D):
    return pl.BlockSpec((_B, width), lambda i: (i, 0))


def _full_spec(shape):
    return pl.BlockSpec(shape, lambda i: (0, 0))


_l1_call = pl.pallas_call(
    _l1_body,
    grid=(_GRID,),
    in_specs=[
        _smem,
        _row_spec(),
        _full_spec((D, D)),
        _full_spec((1, D)),
        _row_spec(1),
        _row_spec(1),
    ],
    out_specs=[_row_spec(), _row_spec()],
    out_shape=[
        jax.ShapeDtypeStruct((N, D), jnp.float32),
        jax.ShapeDtypeStruct((N, D), jnp.float32),
    ],
)

_mid_call = pl.pallas_call(
    _mid_body,
    grid=(_GRID,),
    in_specs=[
        _smem,
        _smem,
        _row_spec(),
        _row_spec(),
        _row_spec(),
        _row_spec(),
        _row_spec(1),
        _row_spec(1),
        _full_spec((D, D)),
        _full_spec((1, D)),
    ],
    out_specs=[_row_spec(), _row_spec()],
    out_shape=[
        jax.ShapeDtypeStruct((N, D), jnp.float32),
        jax.ShapeDtypeStruct((N, D), jnp.float32),
    ],
)

_fin_call = pl.pallas_call(
    _fin_body,
    grid=(_GRID,),
    in_specs=[
        _smem,
        _row_spec(),
        _row_spec(),
        _row_spec(),
        _row_spec(),
        _row_spec(1),
        _row_spec(1),
    ],
    out_specs=_row_spec(),
    out_shape=jax.ShapeDtypeStruct((N, D), jnp.float32),
)


@jax.jit
def kernel(x, W1, b1, W2, b2, g1, g2, edge_index):
    src = edge_index[0].astype(jnp.int32)
    dst = edge_index[1].astype(jnp.int32)
    pad = E_PAD - E
    pidx = jnp.arange(pad, dtype=jnp.int32)
    srcp = jnp.concatenate([src, pidx % N])
    dstp = jnp.concatenate([dst, N + pidx % (AGG_ROWS - N)])
    dst_deg = dstp.reshape(NW, NCHUNK, CH)
    src_agg = srcp.reshape(NCHT, CH)
    dst_agg = dstp.reshape(NCHT, CH)

    degp = _deg_kernel(dst_deg)
    dega = degp[0].reshape(DEG_PAD, 1)
    degb = degp[1].reshape(DEG_PAD, 1)

    zeros_agg = jnp.zeros((AGG_PT, D), jnp.float32)
    b1r = b1.reshape(1, D)
    b2r = b2.reshape(1, D)

    h1, hs1 = _l1_call(g1, x, W1, b1r, dega, degb)
    agg1 = _agg_kernel(hs1, src_agg, dst_agg, zeros_agg)
    h2, hs2 = _mid_call(g1, g2, h1, hs1, agg1[0], agg1[1],
                        dega, degb, W2, b2r)
    agg2 = _agg_kernel(hs2, src_agg, dst_agg, zeros_agg)
    return _fin_call(g2, h2, hs2, agg2[0], agg2[1], dega, degb)


# speedup vs baseline: 1.2432x; 1.0030x over previous
import functools

import jax
import jax.numpy as jnp
from jax import lax
from jax.experimental import pallas as pl
from jax.experimental.pallas import tpu as pltpu
from jax.experimental.pallas import tpu_sc as plsc

N = 10000
E = 320000
D = 128

NC = 2
NS = 16
NW = NC * NS

CH = 128
NCHUNK = 80
EPT = CH * NCHUNK
E_PAD = EPT * NW

AGG_ROWS = 10112
AGG_PT = AGG_ROWS // NS
W0 = 80
W1 = 80
NW1 = NS * W1
NCHT = NS * (W0 + W1)
DEG_PAD = 10240
DEG_PT = DEG_PAD // NS

_mesh = plsc.VectorSubcoreMesh(core_axis_name="c", subcore_axis_name="s")


@functools.partial(
    pl.kernel,
    out_type=jax.ShapeDtypeStruct((NC, DEG_PAD), jnp.float32),
    mesh=_mesh,
    scratch_types=[
        pltpu.VMEM((NCHUNK, CH), jnp.int32),
        pltpu.VMEM((CH,), jnp.float32),
        pltpu.VMEM((DEG_PT,), jnp.float32),
        pltpu.VMEM_SHARED((DEG_PAD,), jnp.float32),
        pltpu.SemaphoreType.DMA,
    ],
)
def _deg_kernel(dst_hbm, out_hbm, dst_v, ones_v, zer_v, deg_sh, sem):
    cid = lax.axis_index("c")
    sid = lax.axis_index("s")
    wid = sid * NC + cid
    for i in range(CH // 16):
        ones_v[pl.ds(i * 16, 16)] = jnp.ones((16,), jnp.float32)
    for i in range(DEG_PT // 16):
        zer_v[pl.ds(i * 16, 16)] = jnp.zeros((16,), jnp.float32)
    pltpu.sync_copy(zer_v, deg_sh.at[pl.ds(sid * DEG_PT, DEG_PT)])
    pltpu.sync_copy(dst_hbm.at[wid], dst_v)
    plsc.subcore_barrier()

    def fire(i, carry):
        pltpu.async_copy(ones_v, deg_sh.at[dst_v.at[i]], sem, add=True)
        return carry

    lax.fori_loop(0, NCHUNK, fire, 0)

    def drain(i, carry):
        pltpu.make_async_copy(ones_v, deg_sh.at[dst_v.at[i]], sem).wait()
        return carry

    lax.fori_loop(0, NCHUNK, drain, 0)
    plsc.subcore_barrier()
    pltpu.sync_copy(
        deg_sh.at[pl.ds(sid * DEG_PT, DEG_PT)],
        out_hbm.at[cid, pl.ds(sid * DEG_PT, DEG_PT)],
    )


@functools.partial(
    pl.kernel,
    out_type=jax.ShapeDtypeStruct((NC, AGG_ROWS, D), jnp.float32),
    mesh=_mesh,
    scratch_types=[
        pltpu.VMEM((W0, CH), jnp.int32),
        pltpu.VMEM((2, CH), jnp.int32),
        pltpu.VMEM((CH, D), jnp.float32),
        pltpu.VMEM((CH, D), jnp.float32),
        pltpu.VMEM_SHARED((AGG_ROWS, D), jnp.float32),
        pltpu.SemaphoreType.DMA,
        pltpu.SemaphoreType.DMA,
        pltpu.SemaphoreType.DMA,
        pltpu.SemaphoreType.DMA,
    ],
)
def _agg_kernel(hs_hbm, src_hbm, dst_hbm, zeros_hbm, out_hbm,
                src_v, dst_v, r0, r1, agg_sh,
                sem0, sem1, semd0, semd1):
    cid = lax.axis_index("c")
    sid = lax.axis_index("s")
    base = jnp.where(cid == 0, NW1 + sid * W0, sid * W1)
    nch = jnp.where(cid == 0, W0, W1)
    pltpu.sync_copy(zeros_hbm, agg_sh.at[pl.ds(sid * AGG_PT, AGG_PT)])

    @pl.when(cid == 0)
    def _():
        pltpu.sync_copy(src_hbm.at[pl.ds(NW1 + sid * W0, W0)],
                        src_v.at[pl.ds(0, W0)])

    @pl.when(cid == 1)
    def _():
        pltpu.sync_copy(src_hbm.at[pl.ds(sid * W1, W1)],
                        src_v.at[pl.ds(0, W1)])

    plsc.subcore_barrier()

    pltpu.async_copy(hs_hbm.at[src_v.at[0]], r0, sem0)
    pltpu.async_copy(dst_hbm.at[base], dst_v.at[0], semd0)

    def pair(i, carry):
        c0 = 2 * i
        pltpu.async_copy(hs_hbm.at[src_v.at[c0 + 1]], r1, sem1)
        pltpu.async_copy(dst_hbm.at[base + c0 + 1], dst_v.at[1], semd1)
        pltpu.make_async_copy(hs_hbm.at[src_v.at[c0]], r0, sem0).wait()
        pltpu.make_async_copy(dst_hbm.at[base + c0], dst_v.at[0],
                              semd0).wait()
        pltpu.sync_copy(r0, agg_sh.at[dst_v.at[0]], add=True)

        @pl.when(c0 + 2 < nch)
        def _():
            pltpu.async_copy(hs_hbm.at[src_v.at[c0 + 2]], r0, sem0)
            pltpu.async_copy(dst_hbm.at[base + c0 + 2], dst_v.at[0], semd0)

        pltpu.make_async_copy(hs_hbm.at[src_v.at[c0 + 1]], r1, sem1).wait()
        pltpu.make_async_copy(dst_hbm.at[base + c0 + 1], dst_v.at[1],
                              semd1).wait()
        pltpu.sync_copy(r1, agg_sh.at[dst_v.at[1]], add=True)
        return carry

    lax.fori_loop(0, nch // 2, pair, 0)
    plsc.subcore_barrier()
    pltpu.sync_copy(
        agg_sh.at[pl.ds(sid * AGG_PT, AGG_PT)],
        out_hbm.at[cid, pl.ds(sid * AGG_PT, AGG_PT)],
    )


def _spow(d_a, e):
    return jnp.where(
        d_a > 0.0, jnp.exp(e * jnp.log(jnp.maximum(d_a, 1e-12))), 0.0
    )


def _l1_body(g1_ref, x_ref, w1_ref, b1_ref, dega_ref, degb_ref, h_ref, hs_ref):
    h = jnp.dot(x_ref[...], w1_ref[...], preferred_element_type=jnp.float32)
    h = h + b1_ref[...]
    a = g1_ref[6]
    d_a = dega_ref[...] + degb_ref[...] + a
    h_ref[...] = h
    hs_ref[...] = h * _spow(d_a, g1_ref[5])


def _mid_body(g1_ref, g2_ref, h1_ref, hs1_ref, a1a_ref, a1b_ref,
              dega_ref, degb_ref, w2_ref, b2_ref, h2_ref, hs2_ref):
    m1, m2, m3 = g1_ref[0], g1_ref[1], g1_ref[2]
    e1, e2, a = g1_ref[3], g1_ref[4], g1_ref[6]
    deg = dega_ref[...] + degb_ref[...]
    d_a = deg + a
    h1 = h1_ref[...]
    agg = a1a_ref[...] + a1b_ref[...] + a * hs1_ref[...]
    out1 = m1 * _spow(d_a, e1) * h1 + m2 * (_spow(d_a, e2) * agg) + m3 * h1
    r = jnp.maximum(out1, 0.0)
    h2 = jnp.dot(r, w2_ref[...], preferred_element_type=jnp.float32)
    h2 = h2 + b2_ref[...]
    d_a2 = deg + g2_ref[6]
    h2_ref[...] = h2
    hs2_ref[...] = h2 * _spow(d_a2, g2_ref[5])


def _fin_body(g2_ref, h2_ref, hs2_ref, a2a_ref, a2b_ref,
              dega_ref, degb_ref, out_ref):
    m1, m2, m3 = g2_ref[0], g2_ref[1], g2_ref[2]
    e1, e2, a = g2_ref[3], g2_ref[4], g2_ref[6]
    d_a = dega_ref[...] + degb_ref[...] + a
    h2 = h2_ref[...]
    agg = a2a_ref[...] + a2b_ref[...] + a * hs2_ref[...]
    out_ref[...] = (
        m1 * _spow(d_a, e1) * h2 + m2 * (_spow(d_a, e2) * agg) + m3 * h2
    )


_B = 1000
_GRID = N // _B

_smem = pl.BlockSpec(memory_space=pltpu.SMEM)


def _row_spec(width=D):
    return pl.BlockSpec((_B, width), lambda i: (i, 0))


def _full_spec(shape):
    return pl.BlockSpec(shape, lambda i: (0, 0))


_l1_call = pl.pallas_call(
    _l1_body,
    grid=(_GRID,),
    in_specs=[
        _smem,
        _row_spec(),
        _full_spec((D, D)),
        _full_spec((1, D)),
        _row_spec(1),
        _row_spec(1),
    ],
    out_specs=[_row_spec(), _row_spec()],
    out_shape=[
        jax.ShapeDtypeStruct((N, D), jnp.float32),
        jax.ShapeDtypeStruct((N, D), jnp.float32),
    ],
)

_mid_call = pl.pallas_call(
    _mid_body,
    grid=(_GRID,),
    in_specs=[
        _smem,
        _smem,
        _row_spec(),
        _row_spec(),
        _row_spec(),
        _row_spec(),
        _row_spec(1),
        _row_spec(1),
        _full_spec((D, D)),
        _full_spec((1, D)),
    ],
    out_specs=[_row_spec(), _row_spec()],
    out_shape=[
        jax.ShapeDtypeStruct((N, D), jnp.float32),
        jax.ShapeDtypeStruct((N, D), jnp.float32),
    ],
)

_fin_call = pl.pallas_call(
    _fin_body,
    grid=(_GRID,),
    in_specs=[
        _smem,
        _row_spec(),
        _row_spec(),
        _row_spec(),
        _row_spec(),
        _row_spec(1),
        _row_spec(1),
    ],
    out_specs=_row_spec(),
    out_shape=jax.ShapeDtypeStruct((N, D), jnp.float32),
)


@jax.jit
def kernel(x, W1, b1, W2, b2, g1, g2, edge_index):
    src = edge_index[0].astype(jnp.int32)
    dst = edge_index[1].astype(jnp.int32)
    pad = E_PAD - E
    pidx = jnp.arange(pad, dtype=jnp.int32)
    srcp = jnp.concatenate([src, pidx % N])
    dstp = jnp.concatenate([dst, N + pidx % (AGG_ROWS - N)])
    dst_deg = dstp.reshape(NW, NCHUNK, CH)
    src_agg = srcp.reshape(NCHT, CH)
    dst_agg = dstp.reshape(NCHT, CH)

    degp = _deg_kernel(dst_deg)
    dega = degp[0].reshape(DEG_PAD, 1)
    degb = degp[1].reshape(DEG_PAD, 1)

    zeros_agg = jnp.zeros((AGG_PT, D), jnp.float32)
    b1r = b1.reshape(1, D)
    b2r = b2.reshape(1, D)

    h1, hs1 = _l1_call(g1, x, W1, b1r, dega, degb)
    agg1 = _agg_kernel(hs1, src_agg, dst_agg, zeros_agg)
    h2, hs2 = _mid_call(g1, g2, h1, hs1, agg1[0], agg1[1],
                        dega, degb, W2, b2r)
    agg2 = _agg_kernel(hs2, src_agg, dst_agg, zeros_agg)
    return _fin_call(g2, h2, hs2, agg2[0], agg2[1], dega, degb)
